# baseline (speedup 1.0000x reference)
import functools

import jax
import jax.numpy as jnp
from jax import lax
from jax.experimental import pallas as pl
from jax.experimental.pallas import tpu as pltpu
from jax.experimental.pallas import tpu_sc as plsc

_NUM_CLASSES = 1000
_BATCH = 16384
_FEAT = 100
_OUT_W = _NUM_CLASSES + _FEAT
_NC = 2
_NS = 16
_NW = _NC * _NS
_ROWS_PER_W = _BATCH // _NW
_R = 64
_CHUNKS = _ROWS_PER_W // _R
_L = 16


def _sc_body(x_hbm, zeros_hbm, out_hbm, ohv):
    wid = lax.axis_index("s") * _NC + lax.axis_index("c")
    base = wid * _ROWS_PER_W

    pltpu.sync_copy(zeros_hbm, ohv)

    lanes = lax.iota(jnp.int32, _L)
    col0 = jnp.full((_L,), _NUM_CLASSES, jnp.int32)
    ones_v = jnp.full((_L,), 1.0, jnp.float32)
    zeros_v = jnp.zeros((_L,), jnp.float32)

    for c in range(_CHUNKS):
        r0 = base + c * _R
        pltpu.sync_copy(x_hbm.at[pl.ds(r0, _R)],
                        ohv.at[:, pl.ds(_NUM_CLASSES, _FEAT)])

        sels = []
        for g in range(_R // _L):
            rid = lanes + (g * _L)
            selv = plsc.load_gather(ohv, [rid, col0])
            sel = selv.astype(jnp.int32)
            mask = (sel >= 0) & (sel < _NUM_CLASSES)
            plsc.store_scatter(ohv, [rid, sel], ones_v, mask=mask)
            sels.append((rid, sel, mask))

        pltpu.sync_copy(ohv, out_hbm.at[pl.ds(r0, _R)])

        for rid, sel, mask in sels:
            plsc.store_scatter(ohv, [rid, sel], zeros_v, mask=mask)


def kernel(x):
    zeros_src = jnp.zeros((_R, _OUT_W), jnp.float32)
    mesh = plsc.VectorSubcoreMesh(core_axis_name="c", subcore_axis_name="s")
    run = functools.partial(
        pl.kernel,
        mesh=mesh,
        out_type=jax.ShapeDtypeStruct((_BATCH, _OUT_W), jnp.float32),
        scratch_types=[
            pltpu.VMEM((_R, _OUT_W), jnp.float32),
        ],
        compiler_params=pltpu.CompilerParams(use_tc_tiling_on_sc=False, needs_layout_passes=False),
    )(_sc_body)
    return run(x, zeros_src)


# speedup vs baseline: 1.1658x; 1.1658x over previous
import functools

import jax
import jax.numpy as jnp
from jax import lax
from jax.experimental import pallas as pl
from jax.experimental.pallas import tpu as pltpu
from jax.experimental.pallas import tpu_sc as plsc

_NUM_CLASSES = 1000
_BATCH = 16384
_FEAT = 100
_OUT_W = _NUM_CLASSES + _FEAT
_NC = 2
_NS = 16
_NW = _NC * _NS
_ROWS_PER_W = _BATCH // _NW
_R = 64
_CHUNKS = _ROWS_PER_W // _R
_L = 16


def _sc_body(x_hbm, zeros_hbm, out_hbm, xv, ohv):
    wid = lax.axis_index("s") * _NC + lax.axis_index("c")
    base = wid * _ROWS_PER_W

    pltpu.sync_copy(zeros_hbm, ohv)

    lanes = lax.iota(jnp.int32, _L)
    ones_v = jnp.full((_L,), 1.0, jnp.float32)
    zeros_v = jnp.zeros((_L,), jnp.float32)
    rids = [lanes + (g * _L) for g in range(_R // _L)]

    def chunk_body(c, carry):
        r0 = pl.multiple_of(base + c * _R, _R)
        pltpu.sync_copy(x_hbm.at[pl.ds(r0, _R)], xv)

        def col_body(j, carry2):
            for rid in rids:
                v = plsc.load_gather(xv, [rid, jnp.full((_L,), j, jnp.int32)])
                plsc.store_scatter(
                    ohv, [rid, jnp.full((_L,), _NUM_CLASSES + j, jnp.int32)], v)
            return carry2

        lax.fori_loop(0, _FEAT, col_body, 0)

        sels = []
        for rid in rids:
            selv = plsc.load_gather(xv, [rid, jnp.zeros((_L,), jnp.int32)])
            sel = selv.astype(jnp.int32)
            mask = (sel >= 0) & (sel < _NUM_CLASSES)
            plsc.store_scatter(ohv, [rid, sel], ones_v, mask=mask)
            sels.append((rid, sel, mask))

        pltpu.sync_copy(ohv, out_hbm.at[pl.ds(r0, _R)])

        for rid, sel, mask in sels:
            plsc.store_scatter(ohv, [rid, sel], zeros_v, mask=mask)
        return carry

    lax.fori_loop(0, _CHUNKS, chunk_body, 0)


def kernel(x):
    zeros_src = jnp.zeros((_R, _OUT_W), jnp.float32)
    mesh = plsc.VectorSubcoreMesh(core_axis_name="c", subcore_axis_name="s")
    run = functools.partial(
        pl.kernel,
        mesh=mesh,
        out_type=jax.ShapeDtypeStruct((_BATCH, _OUT_W), jnp.float32),
        scratch_types=[
            pltpu.VMEM((_R, _FEAT), jnp.float32),
            pltpu.VMEM((_R, _OUT_W), jnp.float32),
        ],
        compiler_params=pltpu.CompilerParams(needs_layout_passes=False),
    )(_sc_body)
    return run(x, zeros_src)


# speedup vs baseline: 8.7534x; 7.5083x over previous
import jax
import jax.numpy as jnp
from jax.experimental import pallas as pl

_NUM_CLASSES = 1000
_BATCH = 16384
_FEAT = 100
_OUT_H = _NUM_CLASSES + _FEAT
_BN = 2048


def _onehot_concat_kernel(xt_ref, o_ref):
    xb = xt_ref[...]
    sel = xb[0:1, :].astype(jnp.int32)
    rows = jax.lax.broadcasted_iota(jnp.int32, (_NUM_CLASSES, _BN), 0)
    oh = (rows == sel).astype(xb.dtype)
    o_ref[...] = jnp.concatenate([oh, xb], axis=0)


def kernel(x):
    xt = x.T
    grid = (_BATCH // _BN,)
    out_t = pl.pallas_call(
        _onehot_concat_kernel,
        grid=grid,
        in_specs=[pl.BlockSpec((_FEAT, _BN), lambda i: (0, i))],
        out_specs=pl.BlockSpec((_OUT_H, _BN), lambda i: (0, i)),
        out_shape=jax.ShapeDtypeStruct((_OUT_H, _BATCH), x.dtype),
    )(xt)
    return out_t.T
